# Initial kernel scaffold; baseline (speedup 1.0000x reference)
import jax
import jax.numpy as jnp
from jax.experimental import pallas as pl


def kernel(x, w, b, gamma, beta):
    raise NotImplementedError("write your pallas kernel here")


# speedup vs baseline: 1.1210x; 1.1210x over previous
import functools

import jax
import jax.numpy as jnp
from jax.experimental import pallas as pl
from jax.experimental.pallas import tpu as pltpu

_LN_EPS = 1e-5


def _ln_kernel(x_ref, w_ref, b_ref, scale_ref, beta_ref, gmat_ref, o_ref):
    d = jnp.dot(x_ref[...], w_ref[...], preferred_element_type=jnp.float32)
    d = d + b_ref[...]
    var = jnp.dot(d * d, gmat_ref[...], preferred_element_type=jnp.float32)
    t = jax.lax.rsqrt(var + _LN_EPS) * scale_ref[...]
    o_ref[...] = jnp.maximum(d * t + beta_ref[...], 0.0).astype(o_ref.dtype)


def _pick_pack(batch, hidden, target_lanes=256):
    p = 1
    while (p * 2 * hidden <= target_lanes) and (batch % (p * 2) == 0):
        p *= 2
    return p


@functools.partial(jax.jit, static_argnames=("row_block",))
def _encode(x, w, b, gamma, beta, *, row_block=1024):
    batch, a_dim = x.shape
    h_dim = w.shape[1]

    pack = _pick_pack(batch, h_dim)
    rows = batch // pack

    w_c = w - jnp.mean(w, axis=1, keepdims=True)
    b_c = b - jnp.mean(b)

    eye = jnp.eye(pack, dtype=w.dtype)
    w_p = jnp.kron(eye, w_c)
    gmat = jnp.kron(eye, jnp.full((h_dim, h_dim), 1.0 / h_dim, dtype=w.dtype))
    b_p = jnp.tile(b_c, pack).reshape(1, pack * h_dim)
    g_p = jnp.tile(gamma, pack).reshape(1, pack * h_dim)
    be_p = jnp.tile(beta, pack).reshape(1, pack * h_dim)

    x_p = x.reshape(rows, pack * a_dim)
    out_struct = jax.ShapeDtypeStruct((rows, pack * h_dim), jnp.float32)

    cost = pl.CostEstimate(
        flops=2 * batch * a_dim * h_dim,
        transcendentals=batch,
        bytes_accessed=4 * (batch * (a_dim + h_dim) + a_dim * h_dim + 3 * h_dim),
    )

    rb = min(row_block, rows)
    rb = max(8, (rb // 8) * 8)
    grid = (pl.cdiv(rows, rb),)
    out_p = pl.pallas_call(
        _ln_kernel,
        out_shape=out_struct,
        grid=grid,
        in_specs=[
            pl.BlockSpec((rb, pack * a_dim), lambda i: (i, 0)),
            pl.BlockSpec((pack * a_dim, pack * h_dim), lambda i: (0, 0)),
            pl.BlockSpec((1, pack * h_dim), lambda i: (0, 0)),
            pl.BlockSpec((1, pack * h_dim), lambda i: (0, 0)),
            pl.BlockSpec((1, pack * h_dim), lambda i: (0, 0)),
            pl.BlockSpec((pack * h_dim, pack * h_dim), lambda i: (0, 0)),
        ],
        out_specs=pl.BlockSpec((rb, pack * h_dim), lambda i: (i, 0)),
        compiler_params=pltpu.CompilerParams(
            dimension_semantics=("parallel",),
        ),
        cost_estimate=cost,
    )(x_p, w_p, b_p, g_p, be_p, gmat)
    return out_p.reshape(batch, h_dim)


def kernel(x, w, b, gamma, beta):
    return _encode(x, w, b, gamma, beta)


# speedup vs baseline: 1.2745x; 1.1369x over previous
import functools

import jax
import jax.numpy as jnp
from jax.experimental import pallas as pl
from jax.experimental.pallas import tpu as pltpu

_LN_EPS = 1e-5


def _ln_kernel(pack, x_ref, w_ref, b_ref, scale_ref, beta_ref, gmat_ref, o_ref):
    rows_in, a_dim = x_ref.shape
    rows = rows_in // pack
    h_dim = o_ref.shape[1]
    x = x_ref[...]
    xp = jnp.concatenate([x[k * rows:(k + 1) * rows, :] for k in range(pack)],
                         axis=1)
    d = jnp.dot(xp, w_ref[...], preferred_element_type=jnp.float32)
    d = d + b_ref[...]
    var = jnp.dot(d * d, gmat_ref[...], preferred_element_type=jnp.float32)
    t = jax.lax.rsqrt(var + _LN_EPS) * scale_ref[...]
    y = jnp.maximum(d * t + beta_ref[...], 0.0).astype(o_ref.dtype)
    for k in range(pack):
        o_ref[k * rows:(k + 1) * rows, :] = y[:, k * h_dim:(k + 1) * h_dim]


def _pick_pack(batch, hidden, target_lanes=256):
    p = 1
    while (p * 2 * hidden <= target_lanes) and (batch % (p * 2) == 0):
        p *= 2
    return p


@functools.partial(jax.jit, static_argnames=("row_block",))
def _encode(x, w, b, gamma, beta, *, row_block=1024):
    batch, a_dim = x.shape
    h_dim = w.shape[1]

    pack = _pick_pack(batch, h_dim)
    rows = batch // pack

    w_c = w - jnp.mean(w, axis=1, keepdims=True)
    b_c = b - jnp.mean(b)

    eye = jnp.eye(pack, dtype=w.dtype)
    w_p = jnp.kron(eye, w_c)
    gmat = jnp.kron(eye, jnp.full((h_dim, h_dim), 1.0 / h_dim, dtype=w.dtype))
    b_p = jnp.tile(b_c, pack).reshape(1, pack * h_dim)
    g_p = jnp.tile(gamma, pack).reshape(1, pack * h_dim)
    be_p = jnp.tile(beta, pack).reshape(1, pack * h_dim)

    cost = pl.CostEstimate(
        flops=2 * batch * a_dim * h_dim,
        transcendentals=batch,
        bytes_accessed=4 * (batch * (a_dim + h_dim) + a_dim * h_dim + 3 * h_dim),
    )

    rb = min(row_block, rows)
    rb = max(8, (rb // 8) * 8)
    grid = (pl.cdiv(rows, rb),)
    body = functools.partial(_ln_kernel, pack)
    out = pl.pallas_call(
        body,
        out_shape=jax.ShapeDtypeStruct((batch, h_dim), jnp.float32),
        grid=grid,
        in_specs=[
            pl.BlockSpec((rb * pack, a_dim), lambda i: (i, 0)),
            pl.BlockSpec((pack * a_dim, pack * h_dim), lambda i: (0, 0)),
            pl.BlockSpec((1, pack * h_dim), lambda i: (0, 0)),
            pl.BlockSpec((1, pack * h_dim), lambda i: (0, 0)),
            pl.BlockSpec((1, pack * h_dim), lambda i: (0, 0)),
            pl.BlockSpec((pack * h_dim, pack * h_dim), lambda i: (0, 0)),
        ],
        out_specs=pl.BlockSpec((rb * pack, h_dim), lambda i: (i, 0)),
        compiler_params=pltpu.CompilerParams(
            dimension_semantics=("parallel",),
        ),
        cost_estimate=cost,
    )(x, w_p, b_p, g_p, be_p, gmat)
    return out


def kernel(x, w, b, gamma, beta):
    return _encode(x, w, b, gamma, beta)


# speedup vs baseline: 5.4022x; 4.2386x over previous
import functools

import jax
import jax.numpy as jnp
from jax.experimental import pallas as pl
from jax.experimental.pallas import tpu as pltpu

_LN_EPS = 1e-5


def _ln_t_kernel(w_ref, gm_ref, b_ref, scale_ref, beta_ref, x_ref, o_ref):
    d = jnp.dot(w_ref[...], x_ref[...], preferred_element_type=jnp.float32)
    d = d + b_ref[...]
    var = jnp.dot(gm_ref[...], d * d, preferred_element_type=jnp.float32)
    t = jax.lax.rsqrt(var + _LN_EPS) * scale_ref[...]
    o_ref[...] = jnp.maximum(d * t + beta_ref[...], 0.0).astype(o_ref.dtype)


@functools.partial(jax.jit, static_argnames=("col_block",))
def _encode(x, w, b, gamma, beta, *, col_block=4096):
    batch, a_dim = x.shape
    h_dim = w.shape[1]

    w_c = w - jnp.mean(w, axis=1, keepdims=True)
    b_c = b - jnp.mean(b)

    wt = w_c.T
    gm = jnp.full((h_dim, h_dim), 1.0 / h_dim, x.dtype)
    b_col = b_c.reshape(h_dim, 1)
    g_col = gamma.reshape(h_dim, 1)
    be_col = beta.reshape(h_dim, 1)

    xt = x.T

    cost = pl.CostEstimate(
        flops=2 * batch * a_dim * h_dim,
        transcendentals=batch,
        bytes_accessed=4 * (batch * (a_dim + h_dim) + a_dim * h_dim + 3 * h_dim),
    )

    cb = min(col_block, batch)
    cb = max(128, (cb // 128) * 128)
    grid = (pl.cdiv(batch, cb),)
    yt = pl.pallas_call(
        _ln_t_kernel,
        out_shape=jax.ShapeDtypeStruct((h_dim, batch), jnp.float32),
        grid=grid,
        in_specs=[
            pl.BlockSpec((h_dim, a_dim), lambda i: (0, 0)),
            pl.BlockSpec((h_dim, h_dim), lambda i: (0, 0)),
            pl.BlockSpec((h_dim, 1), lambda i: (0, 0)),
            pl.BlockSpec((h_dim, 1), lambda i: (0, 0)),
            pl.BlockSpec((h_dim, 1), lambda i: (0, 0)),
            pl.BlockSpec((a_dim, cb), lambda i: (0, i)),
        ],
        out_specs=pl.BlockSpec((h_dim, cb), lambda i: (0, i)),
        compiler_params=pltpu.CompilerParams(
            dimension_semantics=("parallel",),
        ),
        cost_estimate=cost,
    )(wt, gm, b_col, g_col, be_col, xt)
    return yt.T


def kernel(x, w, b, gamma, beta):
    return _encode(x, w, b, gamma, beta)


# speedup vs baseline: 10.1107x; 1.8716x over previous
import functools

import jax
import jax.numpy as jnp
from jax.experimental import pallas as pl
from jax.experimental.pallas import tpu as pltpu

_LN_EPS = 1e-5


def _ln_t_kernel(w_ref, gm_ref, b_ref, scale_ref, beta_ref, x_ref, o_ref):
    d = jnp.dot(w_ref[...], x_ref[...], preferred_element_type=jnp.float32)
    d = d + b_ref[...]
    var = jnp.dot(gm_ref[...], d * d, preferred_element_type=jnp.float32)
    t = jax.lax.rsqrt(var + _LN_EPS) * scale_ref[...]
    o_ref[...] = jnp.maximum(d * t + beta_ref[...], 0.0).astype(o_ref.dtype)


@functools.partial(jax.jit, static_argnames=("col_block",))
def _encode(x, w, b, gamma, beta, *, col_block=16384):
    batch, a_dim = x.shape
    h_dim = w.shape[1]

    w_c = w - jnp.mean(w, axis=1, keepdims=True)
    b_c = b - jnp.mean(b)

    wt = w_c.T
    gm = jnp.full((h_dim, h_dim), 1.0 / h_dim, x.dtype)
    b_col = b_c.reshape(h_dim, 1)
    g_col = gamma.reshape(h_dim, 1)
    be_col = beta.reshape(h_dim, 1)

    xt = x.T

    cost = pl.CostEstimate(
        flops=2 * batch * a_dim * h_dim,
        transcendentals=batch,
        bytes_accessed=4 * (batch * (a_dim + h_dim) + a_dim * h_dim + 3 * h_dim),
    )

    cb = min(col_block, batch)
    cb = max(128, (cb // 128) * 128)
    yt = pl.pallas_call(
        _ln_t_kernel,
        out_shape=jax.ShapeDtypeStruct((h_dim, batch), jnp.float32),
        grid=(pl.cdiv(batch, cb),),
        in_specs=[
            pl.BlockSpec((h_dim, a_dim), lambda i: (0, 0)),
            pl.BlockSpec((h_dim, h_dim), lambda i: (0, 0)),
            pl.BlockSpec((h_dim, 1), lambda i: (0, 0)),
            pl.BlockSpec((h_dim, 1), lambda i: (0, 0)),
            pl.BlockSpec((h_dim, 1), lambda i: (0, 0)),
            pl.BlockSpec((a_dim, cb), lambda i: (0, i)),
        ],
        out_specs=pl.BlockSpec((h_dim, cb), lambda i: (0, i)),
        compiler_params=pltpu.CompilerParams(
            dimension_semantics=("parallel",),
        ),
        cost_estimate=cost,
    )(wt, gm, b_col, g_col, be_col, xt)
    return yt.T


def kernel(x, w, b, gamma, beta):
    return _encode(x, w, b, gamma, beta)


# speedup vs baseline: 11.7020x; 1.1574x over previous
import functools

import jax
import jax.numpy as jnp
from jax.experimental import pallas as pl
from jax.experimental.pallas import tpu as pltpu

_LN_EPS = 1e-5


def _ln_t_kernel(w_ref, gm_ref, b_ref, scale_ref, beta_ref, x_ref, o_ref):
    d = jnp.dot(w_ref[...], x_ref[...], preferred_element_type=jnp.float32)
    d = d + b_ref[...]
    var = jnp.dot(gm_ref[...], d * d, preferred_element_type=jnp.float32)
    t = jax.lax.rsqrt(var + _LN_EPS) * scale_ref[...]
    o_ref[...] = jnp.maximum(d * t + beta_ref[...], 0.0).astype(o_ref.dtype)


@functools.partial(jax.jit, static_argnames=("col_block",))
def _encode(x, w, b, gamma, beta, *, col_block=32768):
    batch, a_dim = x.shape
    h_dim = w.shape[1]

    w_c = w - jnp.mean(w, axis=1, keepdims=True)
    b_c = b - jnp.mean(b)

    wt = w_c.T
    gm = jnp.full((h_dim, h_dim), 1.0 / h_dim, x.dtype)
    b_col = b_c.reshape(h_dim, 1)
    g_col = gamma.reshape(h_dim, 1)
    be_col = beta.reshape(h_dim, 1)

    xt = x.T

    cost = pl.CostEstimate(
        flops=2 * batch * a_dim * h_dim,
        transcendentals=batch,
        bytes_accessed=4 * (batch * (a_dim + h_dim) + a_dim * h_dim + 3 * h_dim),
    )

    cb = min(col_block, batch)
    cb = max(128, (cb // 128) * 128)
    yt = pl.pallas_call(
        _ln_t_kernel,
        out_shape=jax.ShapeDtypeStruct((h_dim, batch), jnp.float32),
        grid=(pl.cdiv(batch, cb),),
        in_specs=[
            pl.BlockSpec((h_dim, a_dim), lambda i: (0, 0)),
            pl.BlockSpec((h_dim, h_dim), lambda i: (0, 0)),
            pl.BlockSpec((h_dim, 1), lambda i: (0, 0)),
            pl.BlockSpec((h_dim, 1), lambda i: (0, 0)),
            pl.BlockSpec((h_dim, 1), lambda i: (0, 0)),
            pl.BlockSpec((a_dim, cb), lambda i: (0, i)),
        ],
        out_specs=pl.BlockSpec((h_dim, cb), lambda i: (0, i)),
        compiler_params=pltpu.CompilerParams(
            dimension_semantics=("parallel",),
        ),
        cost_estimate=cost,
    )(wt, gm, b_col, g_col, be_col, xt)
    return yt.T


def kernel(x, w, b, gamma, beta):
    return _encode(x, w, b, gamma, beta)


# speedup vs baseline: 12.4245x; 1.0617x over previous
import functools

import jax
import jax.numpy as jnp
from jax.experimental import pallas as pl
from jax.experimental.pallas import tpu as pltpu

_LN_EPS = 1e-5


def _ln_t_kernel(w_ref, gm_ref, b_ref, scale_ref, beta_ref, x_ref, o_ref):
    d = jnp.dot(w_ref[...], x_ref[...], preferred_element_type=jnp.float32)
    d = d + b_ref[...]
    var = jnp.dot(gm_ref[...], d * d, preferred_element_type=jnp.float32)
    t = jax.lax.rsqrt(var + _LN_EPS) * scale_ref[...]
    o_ref[...] = jnp.maximum(d * t + beta_ref[...], 0.0).astype(o_ref.dtype)


@functools.partial(jax.jit, static_argnames=("col_block",))
def _encode(x, w, b, gamma, beta, *, col_block=65536):
    batch, a_dim = x.shape
    h_dim = w.shape[1]

    w_c = w - jnp.mean(w, axis=1, keepdims=True)
    b_c = b - jnp.mean(b)

    wt = w_c.T
    gm = jnp.full((h_dim, h_dim), 1.0 / h_dim, x.dtype)
    b_col = b_c.reshape(h_dim, 1)
    g_col = gamma.reshape(h_dim, 1)
    be_col = beta.reshape(h_dim, 1)

    xt = x.T

    cost = pl.CostEstimate(
        flops=2 * batch * a_dim * h_dim,
        transcendentals=batch,
        bytes_accessed=4 * (batch * (a_dim + h_dim) + a_dim * h_dim + 3 * h_dim),
    )

    cb = min(col_block, batch)
    cb = max(128, (cb // 128) * 128)
    yt = pl.pallas_call(
        _ln_t_kernel,
        out_shape=jax.ShapeDtypeStruct((h_dim, batch), jnp.float32),
        grid=(pl.cdiv(batch, cb),),
        in_specs=[
            pl.BlockSpec((h_dim, a_dim), lambda i: (0, 0)),
            pl.BlockSpec((h_dim, h_dim), lambda i: (0, 0)),
            pl.BlockSpec((h_dim, 1), lambda i: (0, 0)),
            pl.BlockSpec((h_dim, 1), lambda i: (0, 0)),
            pl.BlockSpec((h_dim, 1), lambda i: (0, 0)),
            pl.BlockSpec((a_dim, cb), lambda i: (0, i)),
        ],
        out_specs=pl.BlockSpec((h_dim, cb), lambda i: (0, i)),
        compiler_params=pltpu.CompilerParams(
            dimension_semantics=("parallel",),
        ),
        cost_estimate=cost,
    )(wt, gm, b_col, g_col, be_col, xt)
    return yt.T


def kernel(x, w, b, gamma, beta):
    return _encode(x, w, b, gamma, beta)


# speedup vs baseline: 13.2684x; 1.0679x over previous
import functools

import jax
import jax.numpy as jnp
from jax.experimental import pallas as pl
from jax.experimental.pallas import tpu as pltpu

_LN_EPS = 1e-5


def _ln_t_kernel(w_ref, gm_ref, b_ref, beta_ref, x_ref, o_ref):
    d = jnp.dot(w_ref[...], x_ref[...], preferred_element_type=jnp.float32)
    d = d + b_ref[...]
    var = jnp.dot(gm_ref[...], d * d, preferred_element_type=jnp.float32)
    r = jax.lax.rsqrt(var + _LN_EPS)
    o_ref[...] = jnp.maximum(d * r + beta_ref[...], 0.0).astype(o_ref.dtype)


@functools.partial(jax.jit, static_argnames=("col_block",))
def _encode(x, w, b, gamma, beta, *, col_block=65536):
    batch, a_dim = x.shape
    h_dim = w.shape[1]

    w_c = w - jnp.mean(w, axis=1, keepdims=True)
    b_c = b - jnp.mean(b)

    wg = (w_c * gamma[None, :]).T
    bg_col = (b_c * gamma).reshape(h_dim, 1)
    g2 = jnp.maximum(gamma * gamma, jnp.float32(1e-30))
    gm_row = (1.0 / (h_dim * g2)).reshape(1, h_dim)
    be_col = beta.reshape(h_dim, 1)

    xt = x.T

    cost = pl.CostEstimate(
        flops=2 * batch * a_dim * h_dim,
        transcendentals=batch,
        bytes_accessed=4 * (batch * (a_dim + h_dim) + a_dim * h_dim + 3 * h_dim),
    )

    cb = min(col_block, batch)
    cb = max(128, (cb // 128) * 128)
    yt = pl.pallas_call(
        _ln_t_kernel,
        out_shape=jax.ShapeDtypeStruct((h_dim, batch), jnp.float32),
        grid=(pl.cdiv(batch, cb),),
        in_specs=[
            pl.BlockSpec((h_dim, a_dim), lambda i: (0, 0)),
            pl.BlockSpec((1, h_dim), lambda i: (0, 0)),
            pl.BlockSpec((h_dim, 1), lambda i: (0, 0)),
            pl.BlockSpec((h_dim, 1), lambda i: (0, 0)),
            pl.BlockSpec((a_dim, cb), lambda i: (0, i)),
        ],
        out_specs=pl.BlockSpec((h_dim, cb), lambda i: (0, i)),
        compiler_params=pltpu.CompilerParams(
            dimension_semantics=("parallel",),
        ),
        cost_estimate=cost,
    )(wg, gm_row, bg_col, be_col, xt)
    return yt.T


def kernel(x, w, b, gamma, beta):
    return _encode(x, w, b, gamma, beta)
